# baseline (speedup 1.0000x reference)
import jax
import jax.numpy as jnp
from jax import lax
from jax.experimental import pallas as pl
from jax.experimental.pallas import tpu as pltpu
from jax.experimental.pallas import tpu_sc as plsc

VOCAB = 1000
D = 128
SEQ = 50
BATCH = 1024
B_TOTAL = BATCH * SEQ

_info = plsc.get_sparse_core_info()
NC = _info.num_cores
NS = _info.num_subcores
NW = SEQ // 2
CHUNK = 128
NCHUNK = 2 * BATCH // CHUNK
TROWS = 64

NBUF = 6
GDEPTH = 3


def _table_body(w_ref, b_ref, out_ref):
    w = w_ref[...]
    r = lax.broadcasted_iota(jnp.int32, (D, D), 0)
    c = lax.broadcasted_iota(jnp.int32, (D, D), 1)
    eye = jnp.where(r == c, 1.0, 0.0).astype(jnp.float32)
    t = lax.dot_general(
        w, eye,
        dimension_numbers=(((0,), (0,)), ((), ())),
        preferred_element_type=jnp.float32,
    )
    out_ref[...] = t + b_ref[...]


def _build_table(W, b):
    return pl.pallas_call(
        _table_body,
        out_shape=jax.ShapeDtypeStruct((VOCAB, D), jnp.float32),
    )(W, b.reshape(1, D))


def _sc_body(table_hbm, idx_hbm, out_hbm, table_s, idx_v, buf_v, *sems):
    sg = sems[:NBUF]
    ss = sems[NBUF:]
    sid = lax.axis_index("s")
    wid = sid * NC + lax.axis_index("c")

    tbase = jnp.minimum(sid * TROWS, VOCAB - TROWS)
    pltpu.sync_copy(table_hbm.at[pl.ds(tbase, TROWS)],
                    table_s.at[pl.ds(tbase, TROWS)])

    sbase = jnp.minimum(wid, NW - 1) * 2

    @pl.when(wid < NW)
    def _stage_idx():
        hs = [
            pltpu.async_copy(
                idx_hbm.at[sbase + k // 8, pl.ds((k % 8) * CHUNK, CHUNK)],
                idx_v.at[k], sems[2 * NBUF])
            for k in range(NCHUNK)
        ]
        for h in hs:
            h.wait()

    plsc.subcore_barrier()

    @pl.when(wid < NW)
    def _work():
        gh = [None] * NCHUNK
        sh = [None] * NCHUNK
        s_waited = [False] * NCHUNK

        def gather(g):
            gh[g] = pltpu.async_copy(
                table_s.at[idx_v.at[g]], buf_v.at[g % NBUF], sg[g % NBUF])

        def scatter(j):
            sh[j] = pltpu.async_copy(
                buf_v.at[j % NBUF],
                out_hbm.at[pl.ds(sbase * BATCH + j * CHUNK, CHUNK)],
                ss[j % NBUF])

        for g in range(min(GDEPTH, NCHUNK)):
            gather(g)
        for j in range(NCHUNK):
            gh[j].wait()
            scatter(j)
            nxt = j + GDEPTH
            if nxt < NCHUNK:
                prev_user = nxt - NBUF
                if prev_user >= 0:
                    sh[prev_user].wait()
                    s_waited[prev_user] = True
                gather(nxt)
        for j in range(NCHUNK):
            if not s_waited[j]:
                sh[j].wait()


def _sc_lookup(table, idxT):
    mesh = plsc.VectorSubcoreMesh(core_axis_name="c", subcore_axis_name="s")
    k = pl.kernel(
        _sc_body,
        mesh=mesh,
        out_type=jax.ShapeDtypeStruct((B_TOTAL, D), jnp.float32),
        scratch_types=[
            pltpu.VMEM_SHARED((VOCAB, D), jnp.float32),
            pltpu.VMEM((NCHUNK, CHUNK), jnp.int32),
            pltpu.VMEM((NBUF, CHUNK, D), jnp.float32),
        ] + [pltpu.SemaphoreType.DMA] * (2 * NBUF + 1),
        compiler_params=pltpu.CompilerParams(use_tc_tiling_on_sc=True),
    )
    return k(table, idxT)


def kernel(x, W, b):
    bsz, seq = x.shape
    idxT = x.astype(jnp.int32).T
    table = _build_table(W, b)
    out = _sc_lookup(table, idxT)
    return out.reshape(seq, bsz, D).transpose(1, 0, 2)


# speedup vs baseline: 1.0042x; 1.0042x over previous
import jax
import jax.numpy as jnp
from jax import lax
from jax.experimental import pallas as pl
from jax.experimental.pallas import tpu as pltpu
from jax.experimental.pallas import tpu_sc as plsc

VOCAB = 1000
D = 128
SEQ = 50
BATCH = 1024
B_TOTAL = BATCH * SEQ

_info = plsc.get_sparse_core_info()
NC = _info.num_cores
NS = _info.num_subcores
NW = SEQ // 2
CHUNK = 64
NCHUNK = 2 * BATCH // CHUNK
TROWS = 64

NBUF = 8
GDEPTH = 4


def _table_body(w_ref, b_ref, out_ref):
    w = w_ref[...]
    r = lax.broadcasted_iota(jnp.int32, (D, D), 0)
    c = lax.broadcasted_iota(jnp.int32, (D, D), 1)
    eye = jnp.where(r == c, 1.0, 0.0).astype(jnp.float32)
    t = lax.dot_general(
        w, eye,
        dimension_numbers=(((0,), (0,)), ((), ())),
        preferred_element_type=jnp.float32,
    )
    out_ref[...] = t + b_ref[...]


def _build_table(W, b):
    return pl.pallas_call(
        _table_body,
        out_shape=jax.ShapeDtypeStruct((VOCAB, D), jnp.float32),
    )(W, b.reshape(1, D))


def _sc_body(table_hbm, idx_hbm, out_hbm, table_s, idx_v, buf_v, *sems):
    sg = sems[:NBUF]
    ss = sems[NBUF:]
    sid = lax.axis_index("s")
    wid = sid * NC + lax.axis_index("c")

    tbase = jnp.minimum(sid * TROWS, VOCAB - TROWS)
    pltpu.sync_copy(table_hbm.at[pl.ds(tbase, TROWS)],
                    table_s.at[pl.ds(tbase, TROWS)])

    sbase = jnp.minimum(wid, NW - 1) * 2

    @pl.when(wid < NW)
    def _stage_idx():
        hs = [
            pltpu.async_copy(
                idx_hbm.at[sbase + k // 16, pl.ds((k % 16) * CHUNK, CHUNK)],
                idx_v.at[k], sems[2 * NBUF])
            for k in range(NCHUNK)
        ]
        for h in hs:
            h.wait()

    plsc.subcore_barrier()

    @pl.when(wid < NW)
    def _work():
        gh = [None] * NCHUNK
        sh = [None] * NCHUNK
        s_waited = [False] * NCHUNK

        def gather(g):
            gh[g] = pltpu.async_copy(
                table_s.at[idx_v.at[g]], buf_v.at[g % NBUF], sg[g % NBUF])

        def scatter(j):
            sh[j] = pltpu.async_copy(
                buf_v.at[j % NBUF],
                out_hbm.at[pl.ds(sbase * BATCH + j * CHUNK, CHUNK)],
                ss[j % NBUF])

        for g in range(min(GDEPTH, NCHUNK)):
            gather(g)
        for j in range(NCHUNK):
            gh[j].wait()
            scatter(j)
            nxt = j + GDEPTH
            if nxt < NCHUNK:
                prev_user = nxt - NBUF
                if prev_user >= 0:
                    sh[prev_user].wait()
                    s_waited[prev_user] = True
                gather(nxt)
        for j in range(NCHUNK):
            if not s_waited[j]:
                sh[j].wait()


def _sc_lookup(table, idxT):
    mesh = plsc.VectorSubcoreMesh(core_axis_name="c", subcore_axis_name="s")
    k = pl.kernel(
        _sc_body,
        mesh=mesh,
        out_type=jax.ShapeDtypeStruct((B_TOTAL, D), jnp.float32),
        scratch_types=[
            pltpu.VMEM_SHARED((VOCAB, D), jnp.float32),
            pltpu.VMEM((NCHUNK, CHUNK), jnp.int32),
            pltpu.VMEM((NBUF, CHUNK, D), jnp.float32),
        ] + [pltpu.SemaphoreType.DMA] * (2 * NBUF + 1),
        compiler_params=pltpu.CompilerParams(use_tc_tiling_on_sc=True),
    )
    return k(table, idxT)


def kernel(x, W, b):
    bsz, seq = x.shape
    idxT = x.astype(jnp.int32).T
    table = _build_table(W, b)
    out = _sc_lookup(table, idxT)
    return out.reshape(seq, bsz, D).transpose(1, 0, 2)
